# Initial kernel scaffold; baseline (speedup 1.0000x reference)
import jax
import jax.numpy as jnp
from jax.experimental import pallas as pl


def kernel(t, x, edge_index, A_vals):
    raise NotImplementedError("write your pallas kernel here")


# speedup vs baseline: 4.5867x; 4.5867x over previous
import functools

import jax
import jax.numpy as jnp
from jax import lax
from jax.experimental import pallas as pl
from jax.experimental.pallas import tpu as pltpu
from jax.experimental.pallas import tpu_sc as plsc

N_NODES = 10000
N_EDGES = 320000
D = 128
DH = D // 2

NC = 2
NS = 16

ET = N_EDGES // NS
K = 80
NCHUNK = ET // K
N_PAD = 10240
ROWS_PER_TILE = N_PAD // NS

_mesh = plsc.VectorSubcoreMesh(core_axis_name="c", subcore_axis_name="s")


@functools.partial(
    pl.kernel,
    out_type=jax.ShapeDtypeStruct((NC, N_PAD, DH), jnp.float32),
    mesh=_mesh,
    compiler_params=pltpu.CompilerParams(use_tc_tiling_on_sc=False),
    scratch_types=[
        pltpu.VMEM((NCHUNK, K), jnp.int32),
        pltpu.VMEM((NCHUNK, K), jnp.int32),
        pltpu.VMEM((NCHUNK, K), jnp.float32),
        pltpu.VMEM((K, DH), jnp.float32),
        pltpu.VMEM_SHARED((N_PAD, DH), jnp.float32),
        pltpu.SemaphoreType.DMA,
    ],
)
def _sc_spmm(xs_hbm, row_hbm, col_hbm, vals_hbm, out_hbm,
             row_v, col_v, vals_v, rows_v, acc_sh, sem):
    c = lax.axis_index("c")
    s = lax.axis_index("s")

    pltpu.sync_copy(row_hbm.at[s], row_v)
    pltpu.sync_copy(col_hbm.at[s], col_v)
    pltpu.sync_copy(vals_hbm.at[s], vals_v)

    coff = (c * N_NODES).astype(jnp.int32) if hasattr(c, "astype") else c * N_NODES

    def _off_body(j, carry):
        for g in range(K // 16):
            sl = pl.ds(g * 16, 16)
            col_v[j, sl] = col_v[j, sl] + coff
        return carry

    lax.fori_loop(0, NCHUNK, _off_body, 0, unroll=False)

    zero16 = jnp.zeros((16,), jnp.float32)

    def _zero_body(e, carry):
        for v in range(DH // 16):
            rows_v[e, pl.ds(v * 16, 16)] = zero16
        return carry

    lax.fori_loop(0, K, _zero_body, 0, unroll=False)
    base = s * ROWS_PER_TILE
    for i in range(ROWS_PER_TILE // K):
        pltpu.sync_copy(rows_v, acc_sh.at[pl.ds(base + i * K, K)])
    plsc.subcore_barrier()

    def _chunk_body(j, carry):
        pltpu.async_copy(xs_hbm.at[col_v.at[j]], rows_v, sem).wait()

        def _scale_body(g, inner):
            a16 = vals_v[j, pl.ds(g * 16, 16)]
            for l in range(16):
                a = a16[l]
                e = g * 16 + l
                for v in range(DH // 16):
                    sl = pl.ds(v * 16, 16)
                    rows_v[e, sl] = rows_v[e, sl] * a
            return inner

        lax.fori_loop(0, K // 16, _scale_body, 0, unroll=False)
        pltpu.sync_copy(rows_v, acc_sh.at[row_v.at[j]], add=True)
        return carry

    lax.fori_loop(0, NCHUNK, _chunk_body, 0, unroll=False)
    plsc.subcore_barrier()

    pltpu.sync_copy(acc_sh.at[pl.ds(base, ROWS_PER_TILE)],
                    out_hbm.at[c, pl.ds(base, ROWS_PER_TILE)])


def _combine_body(p_ref, o_ref):
    o_ref[...] = jnp.maximum(
        jnp.concatenate([p_ref[0], p_ref[1]], axis=-1), 0.0)


_combine = pl.pallas_call(
    _combine_body,
    out_shape=jax.ShapeDtypeStruct((N_NODES, D), jnp.float32),
    grid=(10,),
    in_specs=[pl.BlockSpec((2, N_NODES // 10, DH), lambda i: (0, i, 0))],
    out_specs=pl.BlockSpec((N_NODES // 10, D), lambda i: (i, 0)),
)


def kernel(t, x, edge_index, A_vals):
    xs = jnp.concatenate([x[:, :DH], x[:, DH:]], axis=0)
    row = edge_index[0].reshape(NS, NCHUNK, K)
    col = edge_index[1].reshape(NS, NCHUNK, K)
    vals = A_vals.reshape(NS, NCHUNK, K)
    partials = _sc_spmm(xs, row, col, vals)
    return _combine(partials)


# speedup vs baseline: 5.8800x; 1.2819x over previous
import functools

import jax
import jax.numpy as jnp
from jax import lax
from jax.experimental import pallas as pl
from jax.experimental.pallas import tpu as pltpu
from jax.experimental.pallas import tpu_sc as plsc

N_NODES = 10000
N_EDGES = 320000
D = 128
DH = D // 2

NC = 2
NS = 16

K = 128
NCHUNK = 159
ET = NCHUNK * K
E_PAD = NS * ET
N_PAD = 10240
ROWS_PER_TILE = N_PAD // NS

_mesh = plsc.VectorSubcoreMesh(core_axis_name="c", subcore_axis_name="s")


@functools.partial(
    pl.kernel,
    out_type=jax.ShapeDtypeStruct((NC, N_PAD, DH), jnp.float32),
    mesh=_mesh,
    compiler_params=pltpu.CompilerParams(use_tc_tiling_on_sc=False),
    scratch_types=[
        pltpu.VMEM((NCHUNK, K), jnp.int32),
        pltpu.VMEM((NCHUNK, K), jnp.int32),
        pltpu.VMEM((NCHUNK, K), jnp.float32),
        pltpu.VMEM((K, DH), jnp.float32),
        pltpu.VMEM((K, DH), jnp.float32),
        pltpu.VMEM((K, DH), jnp.float32),
        pltpu.VMEM_SHARED((N_PAD, DH), jnp.float32),
        pltpu.SemaphoreType.DMA,
        pltpu.SemaphoreType.DMA,
        pltpu.SemaphoreType.DMA,
        pltpu.SemaphoreType.DMA,
        pltpu.SemaphoreType.DMA,
        pltpu.SemaphoreType.DMA,
    ],
)
def _sc_spmm(xs_hbm, row_hbm, col_hbm, vals_hbm, out_hbm,
             row_v, col_v, vals_v, rows0, rows1, rows2, acc_sh,
             sg0, sg1, sg2, ss0, ss1, ss2):
    c = lax.axis_index("c")
    s = lax.axis_index("s")
    bufs = (rows0, rows1, rows2)
    gsems = (sg0, sg1, sg2)
    ssems = (ss0, ss1, ss2)

    pltpu.sync_copy(row_hbm.at[s], row_v)
    pltpu.sync_copy(col_hbm.at[s], col_v)
    pltpu.sync_copy(vals_hbm.at[s], vals_v)

    coff = c * N_NODES

    def _off_body(j, carry):
        for g in range(K // 16):
            sl = pl.ds(g * 16, 16)
            col_v[j, sl] = col_v[j, sl] + coff
        return carry

    lax.fori_loop(0, NCHUNK, _off_body, 0, unroll=False)

    zero16 = jnp.zeros((16,), jnp.float32)

    def _zero_body(e, carry):
        for v in range(DH // 16):
            rows0[e, pl.ds(v * 16, 16)] = zero16
        return carry

    lax.fori_loop(0, K, _zero_body, 0, unroll=False)
    base = s * ROWS_PER_TILE
    for i in range(ROWS_PER_TILE // K):
        pltpu.sync_copy(rows0, acc_sh.at[pl.ds(base + i * K, K)])
    plsc.subcore_barrier()

    def _scale(j, buf):
        def _scale_body(g, inner):
            a16 = vals_v[j, pl.ds(g * 16, 16)]
            for l in range(16):
                a = a16[l]
                e = g * 16 + l
                for v in range(DH // 16):
                    sl = pl.ds(v * 16, 16)
                    buf[e, sl] = buf[e, sl] * a
            return inner

        lax.fori_loop(0, K // 16, _scale_body, 0, unroll=False)

    pltpu.async_copy(xs_hbm.at[col_v.at[0]], rows0, sg0)

    def _outer(i, carry):
        for b in range(3):
            j = 3 * i + b
            buf, gsem, ssem = bufs[b], gsems[b], ssems[b]
            bn = (b + 1) % 3

            @pl.when(j >= 2)
            def _():
                pltpu.make_async_copy(
                    bufs[bn], acc_sh.at[row_v.at[j - 2]], ssems[bn]).wait()

            @pl.when(j + 1 < NCHUNK)
            def _():
                pltpu.async_copy(
                    xs_hbm.at[col_v.at[j + 1]], bufs[bn], gsems[bn])

            pltpu.make_async_copy(xs_hbm.at[col_v.at[j]], buf, gsem).wait()
            _scale(j, buf)
            pltpu.async_copy(buf, acc_sh.at[row_v.at[j]], ssem, add=True)
        return carry

    lax.fori_loop(0, NCHUNK // 3, _outer, 0, unroll=False)

    pltpu.make_async_copy(
        bufs[(NCHUNK - 2) % 3], acc_sh.at[row_v.at[NCHUNK - 2]],
        ssems[(NCHUNK - 2) % 3]).wait()
    pltpu.make_async_copy(
        bufs[(NCHUNK - 1) % 3], acc_sh.at[row_v.at[NCHUNK - 1]],
        ssems[(NCHUNK - 1) % 3]).wait()
    plsc.subcore_barrier()

    pltpu.sync_copy(acc_sh.at[pl.ds(base, ROWS_PER_TILE)],
                    out_hbm.at[c, pl.ds(base, ROWS_PER_TILE)])


def _combine_body(p_ref, o_ref):
    o_ref[...] = jnp.maximum(
        jnp.concatenate([p_ref[0], p_ref[1]], axis=-1), 0.0)


_combine = pl.pallas_call(
    _combine_body,
    out_shape=jax.ShapeDtypeStruct((N_NODES, D), jnp.float32),
    grid=(10,),
    in_specs=[pl.BlockSpec((2, N_NODES // 10, DH), lambda i: (0, i, 0))],
    out_specs=pl.BlockSpec((N_NODES // 10, D), lambda i: (i, 0)),
)


def kernel(t, x, edge_index, A_vals):
    xs = jnp.concatenate([x[:, :DH], x[:, DH:]], axis=0)
    pad = E_PAD - N_EDGES
    zpad_i = jnp.zeros((pad,), jnp.int32)
    row = jnp.concatenate([edge_index[0], zpad_i]).reshape(NS, NCHUNK, K)
    col = jnp.concatenate([edge_index[1], zpad_i]).reshape(NS, NCHUNK, K)
    vals = jnp.concatenate(
        [A_vals, jnp.zeros((pad,), jnp.float32)]).reshape(NS, NCHUNK, K)
    partials = _sc_spmm(xs, row, col, vals)
    return _combine(partials)
